# Initial kernel scaffold; baseline (speedup 1.0000x reference)
import jax
import jax.numpy as jnp
from jax.experimental import pallas as pl


def kernel(x, edge_index, batch, params):
    raise NotImplementedError("write your pallas kernel here")


# speedup vs baseline: 5.3853x; 5.3853x over previous
import functools

import jax
import jax.numpy as jnp
from jax import lax
from jax.experimental import pallas as pl
from jax.experimental.pallas import tpu as pltpu
from jax.experimental.pallas import tpu_sc as plsc

N = 10000
N_PAD = 10240
E = 320000
G = 64
BLK = 512
NB = N_PAD // BLK
RPS = N_PAD // 16

EPW = E // 32
ES_CHUNK = 200
ES_NCHUNK = EPW // ES_CHUNK

EPS = E // 16
CS_CHUNK = 200
CS_NCHUNK = EPS // CS_CHUNK


def _sc_mesh():
    return plsc.VectorSubcoreMesh(core_axis_name="c", subcore_axis_name="s")


def _agg_es_body(use_ones, x_hbm, src_hbm, dst_hbm, zeros_hbm, ones_hbm,
                 out0, out1, src_v, dst_v, rows_v, acc, sem):
    c = lax.axis_index("c")
    s = lax.axis_index("s")
    rows0 = s * RPS
    pltpu.sync_copy(zeros_hbm.at[pl.ds(rows0, RPS)], acc.at[pl.ds(rows0, RPS)])
    if use_ones:
        pltpu.sync_copy(ones_hbm, rows_v)
    plsc.subcore_barrier()
    wid = s * 2 + c
    ebase = wid * EPW

    def chunk(k, carry):
        b = ebase + k * ES_CHUNK
        pltpu.sync_copy(dst_hbm.at[pl.ds(b, ES_CHUNK)], dst_v)
        if not use_ones:
            pltpu.sync_copy(src_hbm.at[pl.ds(b, ES_CHUNK)], src_v)
            pltpu.async_copy(x_hbm.at[src_v], rows_v, sem).wait()
        pltpu.sync_copy(rows_v, acc.at[dst_v], add=True)
        return carry

    lax.fori_loop(0, ES_NCHUNK, chunk, 0)
    plsc.subcore_barrier()

    @pl.when(c == 0)
    def _():
        pltpu.sync_copy(acc.at[pl.ds(rows0, RPS)], out0.at[pl.ds(rows0, RPS)])

    @pl.when(c == 1)
    def _():
        pltpu.sync_copy(acc.at[pl.ds(rows0, RPS)], out1.at[pl.ds(rows0, RPS)])


def _agg_es_call(x, src, dst, zeros, ones, use_ones=False):
    kern = pl.kernel(
        functools.partial(_agg_es_body, use_ones),
        out_type=[jax.ShapeDtypeStruct((N_PAD, 128), jnp.float32)] * 2,
        mesh=_sc_mesh(),
        scratch_types=[
            pltpu.VMEM((ES_CHUNK,), jnp.int32),
            pltpu.VMEM((ES_CHUNK,), jnp.int32),
            pltpu.VMEM((ES_CHUNK, 128), jnp.float32),
            pltpu.VMEM_SHARED((N_PAD, 128), jnp.float32),
            pltpu.SemaphoreType.DMA,
        ],
    )
    return tuple(kern(x, src, dst, zeros, ones))


def _agg_cs_body(x_lo, x_hi, src_hbm, dst_hbm, zeros_hbm,
                 out_lo, out_hi, src_v, dst_v, rows_v, acc, sem):
    c = lax.axis_index("c")
    s = lax.axis_index("s")
    rows0 = s * RPS
    pltpu.sync_copy(zeros_hbm.at[pl.ds(rows0, RPS)], acc.at[pl.ds(rows0, RPS)])
    plsc.subcore_barrier()
    ebase = s * EPS

    def chunk(k, carry):
        b = ebase + k * CS_CHUNK
        pltpu.sync_copy(src_hbm.at[pl.ds(b, CS_CHUNK)], src_v)
        pltpu.sync_copy(dst_hbm.at[pl.ds(b, CS_CHUNK)], dst_v)

        @pl.when(c == 0)
        def _():
            pltpu.async_copy(x_lo.at[src_v], rows_v, sem).wait()

        @pl.when(c == 1)
        def _():
            pltpu.async_copy(x_hi.at[src_v], rows_v, sem).wait()

        pltpu.sync_copy(rows_v, acc.at[dst_v], add=True)
        return carry

    lax.fori_loop(0, CS_NCHUNK, chunk, 0)
    plsc.subcore_barrier()

    @pl.when(c == 0)
    def _():
        pltpu.sync_copy(acc.at[pl.ds(rows0, RPS)], out_lo.at[pl.ds(rows0, RPS)])

    @pl.when(c == 1)
    def _():
        pltpu.sync_copy(acc.at[pl.ds(rows0, RPS)], out_hi.at[pl.ds(rows0, RPS)])


def _agg_cs_call(x_lo, x_hi, src, dst, zeros):
    kern = pl.kernel(
        _agg_cs_body,
        out_type=[jax.ShapeDtypeStruct((N_PAD, 128), jnp.float32)] * 2,
        mesh=_sc_mesh(),
        scratch_types=[
            pltpu.VMEM((CS_CHUNK,), jnp.int32),
            pltpu.VMEM((CS_CHUNK,), jnp.int32),
            pltpu.VMEM((CS_CHUNK, 128), jnp.float32),
            pltpu.VMEM_SHARED((N_PAD, 128), jnp.float32),
            pltpu.SemaphoreType.DMA,
        ],
    )
    return tuple(kern(x_lo, x_hi, src, dst, zeros))


def _mm_body(agg_mode, deg_mode, res_n, has_wres, split_out,
             *refs):
    refs = list(refs)
    a0 = refs.pop(0)
    a1 = refs.pop(0)
    if deg_mode == "pair128":
        dg0 = refs.pop(0)
        dg1 = refs.pop(0)
        d = dg0[:, 0:1] + dg1[:, 0:1]
    else:
        d = refs.pop(0)[:, 0:1]
    w = refs.pop(0)
    b = refs.pop(0)
    res_refs = [refs.pop(0) for _ in range(res_n)]
    wres = refs.pop(0) if has_wres else None
    outs = refs
    scale = 1.0 / jnp.maximum(d, 1.0)
    if agg_mode == "sum":
        a = (a0[...] + a1[...]) * scale
    else:
        a = jnp.concatenate([a0[...], a1[...]], axis=1) * scale
    h = jnp.dot(a, w[...], preferred_element_type=jnp.float32) + b[...]
    if res_n:
        r = (res_refs[0][...] if res_n == 1 else
             jnp.concatenate([res_refs[0][...], res_refs[1][...]], axis=1))
        if has_wres:
            h = h + jnp.dot(r, wres[...], preferred_element_type=jnp.float32)
        else:
            h = h + r
    h = jnp.maximum(h, 0.0)
    oi = 0
    if deg_mode == "pair128":
        outs[oi][...] = jnp.broadcast_to(d, (BLK, 16))
        oi += 1
    if split_out:
        outs[oi][...] = h[:, :128]
        outs[oi + 1][...] = h[:, 128:]
    else:
        outs[oi][...] = h


def _mm_call(agg_mode, agg, deg, w, b, res=None, wres=None):
    a0, a1 = agg
    cin = w.shape[0]
    cout = w.shape[1]
    split_out = cout == 256
    deg_mode = "pair128" if isinstance(deg, tuple) else "c16"
    res_n = 0 if res is None else len(res)
    has_wres = wres is not None

    in_specs = [
        pl.BlockSpec((BLK, 128), lambda i: (i, 0)),
        pl.BlockSpec((BLK, 128), lambda i: (i, 0)),
    ]
    args = [a0, a1]
    if deg_mode == "pair128":
        in_specs += [pl.BlockSpec((BLK, 128), lambda i: (i, 0))] * 2
        args += [deg[0], deg[1]]
    else:
        in_specs.append(pl.BlockSpec((BLK, 16), lambda i: (i, 0)))
        args.append(deg)
    in_specs += [
        pl.BlockSpec((cin, cout), lambda i: (0, 0)),
        pl.BlockSpec((1, cout), lambda i: (0, 0)),
    ]
    args += [w, b.reshape(1, cout)]
    if res_n:
        for r in res:
            in_specs.append(pl.BlockSpec((BLK, r.shape[1]), lambda i: (i, 0)))
            args.append(r)
    if has_wres:
        in_specs.append(pl.BlockSpec(wres.shape, lambda i: (0, 0)))
        args.append(wres)

    out_specs = []
    out_shape = []
    if deg_mode == "pair128":
        out_specs.append(pl.BlockSpec((BLK, 16), lambda i: (i, 0)))
        out_shape.append(jax.ShapeDtypeStruct((N_PAD, 16), jnp.float32))
    if split_out:
        out_specs += [pl.BlockSpec((BLK, 128), lambda i: (i, 0))] * 2
        out_shape += [jax.ShapeDtypeStruct((N_PAD, 128), jnp.float32)] * 2
    else:
        out_specs.append(pl.BlockSpec((BLK, 128), lambda i: (i, 0)))
        out_shape.append(jax.ShapeDtypeStruct((N_PAD, 128), jnp.float32))

    out = pl.pallas_call(
        functools.partial(_mm_body, agg_mode, deg_mode, res_n, has_wres,
                          split_out),
        grid=(NB,),
        in_specs=in_specs,
        out_specs=out_specs,
        out_shape=out_shape,
    )(*args)
    return out


def _final_body(x1, x2l, x2h, oh, a1, a2, a3, ab,
                hw1, hb1, hw2, hb2, out, pooled):
    i = pl.program_id(0)

    @pl.when(i == 0)
    def _():
        pooled[...] = jnp.full((G, 256), -jnp.inf, jnp.float32)

    o = (jnp.dot(x1[...], a1[...], preferred_element_type=jnp.float32)
         + jnp.dot(x2l[...], a2[...], preferred_element_type=jnp.float32)
         + jnp.dot(x2h[...], a3[...], preferred_element_type=jnp.float32))
    ohb = oh[...]
    rows = []
    for g in range(G):
        sel = jnp.where(ohb[:, g:g + 1] > 0.0, o, -jnp.inf)
        rows.append(jnp.max(sel, axis=0, keepdims=True))
    blockmax = jnp.concatenate(rows, axis=0)
    pooled[...] = jnp.maximum(pooled[...], blockmax)

    @pl.when(i == NB - 1)
    def _():
        p = pooled[...] + ab[...]
        h = jnp.maximum(
            jnp.dot(p, hw1[...], preferred_element_type=jnp.float32) + hb1[...],
            0.0)
        out[...] = jnp.dot(h, hw2[...], preferred_element_type=jnp.float32) + hb2[...]


def _final_call(x1, x2, onehot, aggr_w, aggr_b, head):
    x2l, x2h = x2
    a1 = aggr_w[0:128]
    a2 = aggr_w[128:256]
    a3 = aggr_w[256:384]
    hw1, hb1 = head[0]["W"], head[0]["b"].reshape(1, -1)
    hw2, hb2 = head[1]["W"], head[1]["b"].reshape(1, -1)
    in_specs = [
        pl.BlockSpec((BLK, 128), lambda i: (i, 0)),
        pl.BlockSpec((BLK, 128), lambda i: (i, 0)),
        pl.BlockSpec((BLK, 128), lambda i: (i, 0)),
        pl.BlockSpec((BLK, G), lambda i: (i, 0)),
        pl.BlockSpec((128, 256), lambda i: (0, 0)),
        pl.BlockSpec((128, 256), lambda i: (0, 0)),
        pl.BlockSpec((128, 256), lambda i: (0, 0)),
        pl.BlockSpec((1, 256), lambda i: (0, 0)),
        pl.BlockSpec((256, 128), lambda i: (0, 0)),
        pl.BlockSpec((1, 128), lambda i: (0, 0)),
        pl.BlockSpec((128, 16), lambda i: (0, 0)),
        pl.BlockSpec((1, 16), lambda i: (0, 0)),
    ]
    return pl.pallas_call(
        _final_body,
        grid=(NB,),
        in_specs=in_specs,
        out_specs=pl.BlockSpec((G, 16), lambda i: (0, 0)),
        out_shape=jax.ShapeDtypeStruct((G, 16), jnp.float32),
        scratch_shapes=[pltpu.VMEM((G, 256), jnp.float32)],
    )(x1, x2l, x2h, onehot, a1, a2, a3,
      aggr_b.reshape(1, 256), hw1, hb1, hw2, hb2)


def kernel(x, edge_index, batch, params):
    src = edge_index[0]
    dst = edge_index[1]
    xp = jnp.pad(x, ((0, N_PAD - N), (0, 0)))

    zeros128 = jnp.zeros((N_PAD, 128), jnp.float32)
    ones_rows = jnp.ones((ES_CHUNK, 128), jnp.float32)
    deg_pair = _agg_es_call(xp, src, dst, zeros128, ones_rows, use_ones=True)

    def agg128(xfull):
        return _agg_es_call(xfull, src, dst, zeros128, ones_rows)

    def agg256(pair):
        return _agg_cs_call(pair[0], pair[1], src, dst, zeros128)

    cur = xp
    deg = deg_pair
    for p in params["blocks"][0]:
        hout = _mm_call("sum", agg128(cur), deg, p["W1"], p["b1"])
        if isinstance(deg, tuple):
            deg_c, h = hout
            deg = deg_c
        else:
            (h,) = hout
        (cur,) = _mm_call("sum", agg128(h), deg, p["W2"], p["b2"], res=(cur,))
    x1 = cur

    pA, pB = params["blocks"][1]
    h = tuple(_mm_call("sum", agg128(cur), deg, pA["W1"], pA["b1"]))
    cur2 = tuple(_mm_call("cat", agg256(h), deg, pA["W2"], pA["b2"],
                          res=(cur,), wres=pA["Wres"]))
    h = tuple(_mm_call("cat", agg256(cur2), deg, pB["W1"], pB["b1"]))
    cur2 = tuple(_mm_call("cat", agg256(h), deg, pB["W2"], pB["b2"], res=cur2))
    x2 = cur2

    batch_pad = jnp.pad(batch, (0, N_PAD - N), constant_values=G)
    onehot = (batch_pad[:, None] == jnp.arange(G)[None, :]).astype(jnp.float32)
    return _final_call(x1, x2, onehot, params["aggr_W"], params["aggr_b"],
                       params["head"])
